# Initial kernel scaffold; baseline (speedup 1.0000x reference)
import jax
import jax.numpy as jnp
from jax.experimental import pallas as pl


def kernel(x_inp, W_emb, b_emb, W1, b1, W2, b2, W3, b3):
    raise NotImplementedError("write your pallas kernel here")


# speedup vs baseline: 20.2110x; 20.2110x over previous
import functools

import jax
import jax.numpy as jnp
from jax import lax
from jax.experimental import pallas as pl
from jax.experimental.pallas import tpu as pltpu
from jax.experimental.pallas import tpu_sc as plsc

_N = 1000
_D = 3
_H = 32
_BR = 8
_G = _N // _BR

_PT = 400
_RT = _PT * _H
_PS = _N - _PT
_CI = 40

_NW = 32
_RPW = _PS * _H // _NW
_CR = 40
_NCH = _RPW // _CR


def _emb_kernel(w_ref, b_ref, wcat_ref, bias_ref, ab_ref):
    rs = jnp.sum(w_ref[...], axis=1, keepdims=True)
    h = rs.reshape(_CI, _H) + b_ref[...]
    ab_ref[...] = (
        jnp.dot(h, wcat_ref[...], preferred_element_type=jnp.float32)
        + bias_ref[...]
    )


def _proj_kernel(rs_ref, b_ref, wcat_ref, bias_ref, ab_ref):
    rs = jnp.sum(rs_ref[...], axis=1, keepdims=True)
    h = rs.reshape(_PS, _H) + b_ref[...]
    ab_ref[...] = (
        jnp.dot(h, wcat_ref[...], preferred_element_type=jnp.float32)
        + bias_ref[...]
    )


def _sc_reduce_body(w_hbm, rs_hbm, buf0, buf1, out_v, sem0, sem1):
    wid = lax.axis_index("s") * 2 + lax.axis_index("c")
    base = _RT + wid * _RPW
    bufs = (buf0, buf1)
    sems = (sem0, sem1)
    lane_hi = lax.iota(jnp.int32, 16) >= 8
    zero16 = jnp.zeros((16,), jnp.float32)

    descs = [None] * (_NCH + 1)
    descs[0] = pltpu.async_copy(w_hbm.at[pl.ds(base, _CR)], bufs[0], sems[0])
    for c in range(_NCH):
        if c + 1 < _NCH:
            descs[c + 1] = pltpu.async_copy(
                w_hbm.at[pl.ds(base + (c + 1) * _CR, _CR)],
                bufs[(c + 1) % 2], sems[(c + 1) % 2])
        descs[c].wait()
        buf = bufs[c % 2]

        def row_body(r, _, buf=buf, c0=c * _CR):
            def k_body(k, a):
                return a + buf[r, pl.ds(k * 16, 16)]
            acc = lax.fori_loop(0, 62, k_body, zero16)
            tail = buf[r, pl.ds(984, 16)]
            acc = acc + jnp.where(lane_hi, tail, zero16)
            out_v[pl.ds((c0 + r) * 16, 16)] = acc
            return 0

        lax.fori_loop(0, _CR, row_body, 0)
    pltpu.sync_copy(out_v, rs_hbm.at[wid])


def _sc_rowsums(w_emb):
    mesh = plsc.VectorSubcoreMesh(core_axis_name="c", subcore_axis_name="s")
    kfn = functools.partial(
        pl.kernel, mesh=mesh,
        out_type=jax.ShapeDtypeStruct((_NW, _RPW * 16), jnp.float32),
        scratch_types=[
            pltpu.VMEM((_CR, _N), jnp.float32),
            pltpu.VMEM((_CR, _N), jnp.float32),
            pltpu.VMEM((_RPW * 16,), jnp.float32),
            pltpu.SemaphoreType.DMA,
            pltpu.SemaphoreType.DMA,
        ],
    )(_sc_reduce_body)
    return kfn(w_emb)


def _silu(v):
    ph = 0.5 * v
    return ph * jnp.tanh(ph) + ph


def _pair_kernel(x_ref, xt_ref, ai_ref, bj_ref, w1c_ref, w2_ref, b2_ref,
                 w3_ref, b3_ref, out_ref):
    xi = x_ref[...]
    xt = xt_ref[...]
    xprod = (xi[:, 0:1] * xt[0:1, :]
             + xi[:, 1:2] * xt[1:2, :]
             + xi[:, 2:3] * xt[2:3, :])
    ni = jnp.sum(xi * xi, axis=1, keepdims=True)
    nj = jnp.sum(xt * xt, axis=0, keepdims=True)
    sqrd = jnp.maximum(ni + nj - 2.0 * xprod, 0.0)

    sqrd_s = jnp.tile(sqrd.astype(jnp.bfloat16), (_H, 1))
    pre1 = ai_ref[0] + bj_ref[...] + w1c_ref[...] * sqrd_s
    t1 = _silu(pre1)
    t2p = jax.lax.dot_general(
        w2_ref[...], t1,
        dimension_numbers=(((1,), (0,)), ((), ())),
        preferred_element_type=jnp.float32)
    t2 = _silu(t2p + b2_ref[...])
    tw = t2 * w3_ref[...]
    t = jnp.sum(tw.reshape(_H, _BR, _N), axis=0) + b3_ref[...]

    rinv = 1.0 / (jnp.sqrt(sqrd) + 1.0)
    s = t * rinv
    s1 = jnp.sum(s, axis=1, keepdims=True)
    s2 = jnp.concatenate(
        [jnp.sum(s * xt[d:d + 1, :], axis=1, keepdims=True) for d in range(_D)],
        axis=1)
    out_ref[...] = xi * (1.0 - s1) + s2


def kernel(x_inp, W_emb, b_emb, W1, b1, W2, b2, W3, b3):
    f32 = jnp.float32
    b_r = b_emb.reshape(_N, _H)
    wcat = jnp.concatenate([W1[:, :_H].T, W1[:, _H:2 * _H].T], axis=1)
    bias64 = jnp.concatenate([b1, jnp.zeros_like(b1)])[None, :]

    rs_sc = _sc_rowsums(W_emb)

    ab1 = pl.pallas_call(
        _emb_kernel,
        grid=(_PT // _CI,),
        in_specs=[
            pl.BlockSpec((_CI * _H, _N), lambda g: (g, 0)),
            pl.BlockSpec((_CI, _H), lambda g: (g, 0)),
            pl.BlockSpec((_H, 2 * _H), lambda g: (0, 0)),
            pl.BlockSpec((1, 2 * _H), lambda g: (0, 0)),
        ],
        out_specs=pl.BlockSpec((_CI, 2 * _H), lambda g: (g, 0)),
        out_shape=jax.ShapeDtypeStruct((_PT, 2 * _H), f32),
        compiler_params=pltpu.CompilerParams(
            dimension_semantics=("parallel",)),
    )(W_emb, b_r, wcat, bias64)

    rs2d = rs_sc.reshape(_PS * _H, 16)
    ab2 = pl.pallas_call(
        _proj_kernel,
        grid=(1,),
        in_specs=[
            pl.BlockSpec((_PS * _H, 16), lambda g: (0, 0)),
            pl.BlockSpec((_PS, _H), lambda g: (0, 0)),
            pl.BlockSpec((_H, 2 * _H), lambda g: (0, 0)),
            pl.BlockSpec((1, 2 * _H), lambda g: (0, 0)),
        ],
        out_specs=pl.BlockSpec((_PS, 2 * _H), lambda g: (0, 0)),
        out_shape=jax.ShapeDtypeStruct((_PS, 2 * _H), f32),
    )(rs2d, b_r[_PT:], wcat, bias64)

    ab = jnp.concatenate([ab1, ab2], axis=0)

    abt = ab.T
    ai_t, bj_t = abt[:_H], abt[_H:]
    ai_s = ai_t.reshape(_H, _G, _BR).transpose(0, 2, 1).reshape(_H * _BR, _G)
    ai_s = ai_s.T.reshape(_G, _H * _BR, 1).astype(jnp.bfloat16)
    bj_s = jnp.repeat(bj_t, _BR, axis=0).astype(jnp.bfloat16)
    w1c_s = jnp.repeat(W1[:, 2 * _H], _BR)[:, None].astype(jnp.bfloat16)
    b2_s = jnp.repeat(b2, _BR)[:, None]
    w3_s = jnp.repeat(W3[0], _BR)[:, None]
    b3_s = b3.reshape(1, 1)
    w2_bd = jnp.kron(W2, jnp.eye(_BR, dtype=W2.dtype)).astype(jnp.bfloat16)

    x = x_inp.reshape(_N, _D)
    xt = x.T

    out = pl.pallas_call(
        _pair_kernel,
        grid=(_G,),
        in_specs=[
            pl.BlockSpec((_BR, _D), lambda g: (g, 0)),
            pl.BlockSpec((_D, _N), lambda g: (0, 0)),
            pl.BlockSpec((1, _H * _BR, 1), lambda g: (g, 0, 0)),
            pl.BlockSpec((_H * _BR, _N), lambda g: (0, 0)),
            pl.BlockSpec((_H * _BR, 1), lambda g: (0, 0)),
            pl.BlockSpec((_H * _BR, _H * _BR), lambda g: (0, 0)),
            pl.BlockSpec((_H * _BR, 1), lambda g: (0, 0)),
            pl.BlockSpec((_H * _BR, 1), lambda g: (0, 0)),
            pl.BlockSpec((1, 1), lambda g: (0, 0)),
        ],
        out_specs=pl.BlockSpec((_BR, _D), lambda g: (g, 0)),
        out_shape=jax.ShapeDtypeStruct((_N, _D), f32),
        compiler_params=pltpu.CompilerParams(
            dimension_semantics=("parallel",)),
    )(x, xt, ai_s, bj_s, w1c_s, w2_bd, b2_s, w3_s, b3_s)

    x_out = jnp.where(jnp.any(jnp.isnan(out)), jnp.zeros_like(out), out)
    return x_out.reshape(_N * _D)


# speedup vs baseline: 32.8802x; 1.6268x over previous
import jax
import jax.numpy as jnp
from jax.experimental import pallas as pl
from jax.experimental.pallas import tpu as pltpu

_N = 1000
_D = 3
_H = 32
_BR = 8
_G = _N // _BR
_CI = 40


def _emb_kernel(w_ref, b_ref, wcat_ref, bias_ref, ab_ref):
    h = jnp.sum(w_ref[...], axis=2) + b_ref[...]
    ab_ref[...] = (
        jnp.dot(h, wcat_ref[...], preferred_element_type=jnp.float32)
        + bias_ref[...]
    )


def _silu(v):
    ph = 0.5 * v
    return ph * jnp.tanh(ph) + ph


def _pair_kernel(x_ref, xt_ref, ai_ref, bj_ref, w1c_ref, w2_ref, b2_ref,
                 w3_ref, b3_ref, out_ref):
    xi = x_ref[...]
    xt = xt_ref[...]
    xprod = (xi[:, 0:1] * xt[0:1, :]
             + xi[:, 1:2] * xt[1:2, :]
             + xi[:, 2:3] * xt[2:3, :])
    ni = jnp.sum(xi * xi, axis=1, keepdims=True)
    nj = jnp.sum(xt * xt, axis=0, keepdims=True)
    sqrd = jnp.maximum(ni + nj - 2.0 * xprod, 0.0)

    sqrd_s = jnp.tile(sqrd.astype(jnp.bfloat16), (_H, 1))
    pre1 = ai_ref[0] + bj_ref[...] + w1c_ref[...] * sqrd_s
    t1 = _silu(pre1)
    dn = (((1,), (0,)), ((), ()))
    t2p = jnp.concatenate([
        jax.lax.dot_general(w2_ref[...], t1[:, :512], dimension_numbers=dn,
                            preferred_element_type=jnp.float32),
        jax.lax.dot_general(w2_ref[...], t1[:, 512:], dimension_numbers=dn,
                            preferred_element_type=jnp.float32),
    ], axis=1)
    tw = _silu((t2p + b2_ref[...]).astype(jnp.bfloat16)) * w3_ref[...]
    t = (jnp.sum(tw.reshape(_H, _BR, _N).astype(jnp.float32), axis=0)
         + b3_ref[...])

    rinv = 1.0 / (jnp.sqrt(sqrd) + 1.0)
    s = t * rinv
    s1 = jnp.sum(s, axis=1, keepdims=True)
    s2 = jnp.concatenate(
        [jnp.sum(s * xt[d:d + 1, :], axis=1, keepdims=True) for d in range(_D)],
        axis=1)
    out_ref[...] = xi * (1.0 - s1) + s2


def kernel(x_inp, W_emb, b_emb, W1, b1, W2, b2, W3, b3):
    f32 = jnp.float32
    w_r = W_emb.reshape(_N, _H, _N)
    b_r = b_emb.reshape(_N, _H)
    wcat = jnp.concatenate([W1[:, :_H].T, W1[:, _H:2 * _H].T], axis=1)
    bias64 = jnp.concatenate([b1, jnp.zeros_like(b1)])[None, :]
    ab = pl.pallas_call(
        _emb_kernel,
        grid=(_N // _CI,),
        in_specs=[
            pl.BlockSpec((_CI, _H, _N), lambda g: (g, 0, 0)),
            pl.BlockSpec((_CI, _H), lambda g: (g, 0)),
            pl.BlockSpec((_H, 2 * _H), lambda g: (0, 0)),
            pl.BlockSpec((1, 2 * _H), lambda g: (0, 0)),
        ],
        out_specs=pl.BlockSpec((_CI, 2 * _H), lambda g: (g, 0)),
        out_shape=jax.ShapeDtypeStruct((_N, 2 * _H), f32),
        compiler_params=pltpu.CompilerParams(
            dimension_semantics=("parallel",)),
    )(w_r, b_r, wcat, bias64)

    abt = ab.T
    ai_t, bj_t = abt[:_H], abt[_H:]
    ai_s = ai_t.reshape(_H, _G, _BR).transpose(0, 2, 1).reshape(_H * _BR, _G)
    ai_s = ai_s.T.reshape(_G, _H * _BR, 1).astype(jnp.bfloat16)
    bj_s = jnp.repeat(bj_t, _BR, axis=0).astype(jnp.bfloat16)
    w1c_s = jnp.repeat(W1[:, 2 * _H], _BR)[:, None].astype(jnp.bfloat16)
    b2_s = jnp.repeat(b2, _BR)[:, None]
    w3_s = jnp.repeat(W3[0], _BR)[:, None].astype(jnp.bfloat16)
    b3_s = b3.reshape(1, 1)
    w2_bd = jnp.kron(W2, jnp.eye(_BR, dtype=W2.dtype)).astype(jnp.bfloat16)

    x = x_inp.reshape(_N, _D)
    xt = x.T

    out = pl.pallas_call(
        _pair_kernel,
        grid=(_G,),
        in_specs=[
            pl.BlockSpec((_BR, _D), lambda g: (g, 0)),
            pl.BlockSpec((_D, _N), lambda g: (0, 0)),
            pl.BlockSpec((1, _H * _BR, 1), lambda g: (g, 0, 0)),
            pl.BlockSpec((_H * _BR, _N), lambda g: (0, 0)),
            pl.BlockSpec((_H * _BR, 1), lambda g: (0, 0)),
            pl.BlockSpec((_H * _BR, _H * _BR), lambda g: (0, 0)),
            pl.BlockSpec((_H * _BR, 1), lambda g: (0, 0)),
            pl.BlockSpec((_H * _BR, 1), lambda g: (0, 0)),
            pl.BlockSpec((1, 1), lambda g: (0, 0)),
        ],
        out_specs=pl.BlockSpec((_BR, _D), lambda g: (g, 0)),
        out_shape=jax.ShapeDtypeStruct((_N, _D), f32),
        compiler_params=pltpu.CompilerParams(
            dimension_semantics=("parallel",)),
    )(x, xt, ai_s, bj_s, w1c_s, w2_bd, b2_s, w3_s, b3_s)

    x_out = jnp.where(jnp.any(jnp.isnan(out)), jnp.zeros_like(out), out)
    return x_out.reshape(_N * _D)
